# baseline (speedup 1.0000x reference)
import functools

import jax
import jax.numpy as jnp
from jax import lax
from jax.experimental import pallas as pl
from jax.experimental.pallas import tpu as pltpu
from jax.experimental.pallas import tpu_sc as plsc

_B = 16384
_D = 1000
_NC = 2
_NS = 16
_NW = _NC * _NS
_RW = _B // _NW
_C = 128
_NCH = _RW // _C

_mesh = plsc.VectorSubcoreMesh(core_axis_name="c", subcore_axis_name="s")


@functools.partial(
    pl.kernel,
    mesh=_mesh,
    out_type=jax.ShapeDtypeStruct((_D, _B), jnp.float32),
    scratch_types=[
        pltpu.VMEM((_RW,), jnp.int32),
        pltpu.VMEM((_D, _C), jnp.float32),
        pltpu.SemaphoreType.DMA,
    ],
    compiler_params=pltpu.CompilerParams(needs_layout_passes=False),
)
def _sc_onehot_t(x_hbm, out_hbm, idx_v, buf, sem):
    wid = lax.axis_index("s") * _NC + lax.axis_index("c")
    base = wid * _RW
    idx_cp = pltpu.async_copy(x_hbm.at[pl.ds(base, _RW)], idx_v, sem)

    zeros = jnp.zeros((16,), jnp.float32)

    def zero_body(i, carry):
        r = i * 4
        for dr in range(4):
            for k in range(_C // 16):
                buf[r + dr, pl.ds(k * 16, 16)] = zeros
        return carry

    lax.fori_loop(0, _D // 4, zero_body, 0)
    idx_cp.wait()

    ones = jnp.ones((16,), jnp.float32)
    col16 = lax.broadcasted_iota(jnp.int32, (16,), 0)
    for c in range(_NCH):
        groups = []
        for g in range(_C // 16):
            cols = col16 + (g * 16)
            cls = idx_v[pl.ds(c * _C + g * 16, 16)]
            plsc.store_scatter(buf, [cls, cols], ones)
            groups.append((cls, cols))
        pltpu.sync_copy(buf, out_hbm.at[:, pl.ds(base + c * _C, _C)])
        if c + 1 < _NCH:
            for cls, cols in groups:
                plsc.store_scatter(buf, [cls, cols], zeros)


def kernel(x):
    x = x.reshape(_B).astype(jnp.int32)
    return _sc_onehot_t(x).T


def build():
    return kernel, (jax.ShapeDtypeStruct((_B, 1), jnp.int32),)


# speedup vs baseline: 1.0102x; 1.0102x over previous
import functools

import jax
import jax.numpy as jnp
from jax import lax
from jax.experimental import pallas as pl
from jax.experimental.pallas import tpu as pltpu
from jax.experimental.pallas import tpu_sc as plsc

_B = 16384
_D = 1000
_NC = 2
_NS = 16
_NW = _NC * _NS
_RW = _B // _NW
_C = 128
_NCH = _RW // _C

_mesh = plsc.VectorSubcoreMesh(core_axis_name="c", subcore_axis_name="s")


@functools.partial(
    pl.kernel,
    mesh=_mesh,
    out_type=jax.ShapeDtypeStruct((_D, _B), jnp.float32),
    scratch_types=[
        pltpu.VMEM((_RW,), jnp.int32),
        pltpu.VMEM((_D, _C), jnp.float32),
        pltpu.SemaphoreType.DMA,
    ],
    compiler_params=pltpu.CompilerParams(needs_layout_passes=False),
)
def _sc_onehot_t(x_hbm, out_hbm, idx_v, buf, sem):
    wid = lax.axis_index("s") * _NC + lax.axis_index("c")
    base = wid * _RW
    idx_cp = pltpu.async_copy(x_hbm.at[pl.ds(base, _RW)], idx_v, sem)

    zeros = jnp.zeros((16,), jnp.float32)

    def zero_body(r, carry):
        for k in range(_C // 16):
            buf[r, pl.ds(k * 16, 16)] = zeros
        return carry

    lax.fori_loop(0, _D, zero_body, 0)
    idx_cp.wait()

    ones = jnp.ones((16,), jnp.float32)
    col16 = lax.broadcasted_iota(jnp.int32, (16,), 0)
    for c in range(_NCH):
        groups = []
        for g in range(_C // 16):
            cols = col16 + (g * 16)
            cls = idx_v[pl.ds(c * _C + g * 16, 16)]
            plsc.store_scatter(buf, [cls, cols], ones)
            groups.append((cls, cols))
        pltpu.sync_copy(buf, out_hbm.at[:, pl.ds(base + c * _C, _C)])
        if c + 1 < _NCH:
            for cls, cols in groups:
                plsc.store_scatter(buf, [cls, cols], zeros)


def kernel(x):
    x = x.reshape(_B).astype(jnp.int32)
    return _sc_onehot_t(x).T


# speedup vs baseline: 1.0394x; 1.0289x over previous
import functools

import jax
import jax.numpy as jnp
from jax import lax
from jax.experimental import pallas as pl
from jax.experimental.pallas import tpu as pltpu
from jax.experimental.pallas import tpu_sc as plsc

_B = 16384
_D = 1000
_NC = 2
_NS = 16
_NW = _NC * _NS
_RW = _B // _NW
_C = 128
_NCH = _RW // _C

_mesh = plsc.VectorSubcoreMesh(core_axis_name="c", subcore_axis_name="s")


@functools.partial(
    pl.kernel,
    mesh=_mesh,
    out_type=jax.ShapeDtypeStruct((_D, _B), jnp.float32),
    scratch_types=[
        pltpu.VMEM((_RW,), jnp.int32),
        pltpu.VMEM((_D, _C), jnp.float32),
        pltpu.SemaphoreType.DMA,
    ],
    compiler_params=pltpu.CompilerParams(needs_layout_passes=False),
)
def _sc_onehot_t(x_hbm, out_hbm, idx_v, buf, sem):
    wid = lax.axis_index("s") * _NC + lax.axis_index("c")
    base = wid * _RW
    idx_cp = pltpu.async_copy(x_hbm.at[pl.ds(base, _RW)], idx_v, sem)

    zeros = jnp.zeros((16,), jnp.float32)

    def zero_body(r, carry):
        for k in range(_C // 16):
            buf[r, pl.ds(k * 16, 16)] = zeros
        return carry

    lax.fori_loop(0, _D, zero_body, 0)
    idx_cp.wait()

    ones = jnp.ones((16,), jnp.float32)
    col16 = lax.broadcasted_iota(jnp.int32, (16,), 0)

    def chunk_body(c, carry):
        col0 = pl.multiple_of(base + c * _C, 128)
        groups = []
        for g in range(_C // 16):
            cols = col16 + (g * 16)
            cls = idx_v[pl.ds(c * _C + g * 16, 16)]
            plsc.store_scatter(buf, [cls, cols], ones)
            groups.append((cls, cols))
        pltpu.sync_copy(buf, out_hbm.at[:, pl.ds(col0, _C)])
        for cls, cols in groups:
            plsc.store_scatter(buf, [cls, cols], zeros)
        return carry

    lax.fori_loop(0, _NCH, chunk_body, 0)


def kernel(x):
    x = x.reshape(_B).astype(jnp.int32)
    return _sc_onehot_t(x).T


def build():
    return kernel, (jax.ShapeDtypeStruct((_B, 1), jnp.int32),)
